# baseline (speedup 1.0000x reference)
import functools

import jax
import jax.numpy as jnp
import numpy as np
from jax import lax
from jax.experimental import pallas as pl
from jax.experimental.pallas import tpu as pltpu
from jax.experimental.pallas import tpu_sc as plsc

B, L, D = 1024, 20, 10000
VOCAB, NUM_CLASSES, NGRAM_N = 1000, 100, 3

NC, NS = 2, 16
NW = NC * NS
BPW = B // NW
RB = 8

NCHUNK = 5
DP = 10240
DC = DP // NCHUNK
HALO = NGRAM_N - 1
W = DC + 128

_NT = L - (NGRAM_N - 1)


def _build_haloed_table(id_weight):
    cols = np.arange(W)[None, :] + (np.arange(NCHUNK) * DC)[:, None] - HALO
    cols = np.where(cols < 0, cols + D, cols)
    wz = jnp.pad(id_weight, ((0, 0), (0, int(cols.max()) + 1 - D)))
    th = jnp.take(wz, jnp.asarray(cols.reshape(-1)), axis=1)
    return th.reshape(VOCAB * NCHUNK, W)


def _sc_encode(table_h, x):
    mesh = plsc.VectorSubcoreMesh(
        core_axis_name="c", subcore_axis_name="s", num_cores=NC, num_subcores=NS
    )

    @functools.partial(
        pl.kernel,
        out_type=jax.ShapeDtypeStruct((B, DP), jnp.float32),
        mesh=mesh,
        compiler_params=pltpu.CompilerParams(use_tc_tiling_on_sc=False),
        scratch_types=[
            pltpu.VMEM((BPW, L), jnp.int32),
            pltpu.VMEM((L,), jnp.int32),
            pltpu.VMEM((L, W), jnp.float32),
            pltpu.VMEM((RB, DC), jnp.float32),
            pltpu.SemaphoreType.DMA,
        ],
    )
    def enc_kernel(table_hbm, x_hbm, enc_hbm, xw, idxv, buf, acc, sem):
        wid = lax.axis_index("s") * NC + lax.axis_index("c")
        base_b = wid * BPW
        pltpu.sync_copy(x_hbm.at[pl.ds(base_b, BPW)], xw)

        def body_grp(i8, carry):
            def body_c(c, carry2):
                def body_r(r, carry3):
                    i = i8 * RB + r
                    idxv[pl.ds(0, 16)] = xw[i, pl.ds(0, 16)] * NCHUNK + c
                    idxv[pl.ds(4, 16)] = xw[i, pl.ds(4, 16)] * NCHUNK + c
                    pltpu.async_copy(table_hbm.at[idxv], buf, sem).wait()

                    for g in range(DC // 16):
                        base = g * 16
                        a = buf[0, pl.ds(base, 16)]
                        a = a * buf[1, pl.ds(base + 1, 16)]
                        a = a * buf[2, pl.ds(base + 2, 16)]
                        acc[r, pl.ds(base, 16)] = a

                    def tbody(t, carry4):
                        for g in range(DC // 16):
                            base = g * 16
                            v = buf[t, pl.ds(base, 16)]
                            v = v * buf[t + 1, pl.ds(base + 1, 16)]
                            v = v * buf[t + 2, pl.ds(base + 2, 16)]
                            acc[r, pl.ds(base, 16)] = acc[r, pl.ds(base, 16)] + v
                        return carry4

                    lax.fori_loop(1, _NT, tbody, 0)
                    return carry3

                lax.fori_loop(0, RB, body_r, 0)
                row0 = pl.multiple_of(base_b + i8 * RB, RB)
                col0 = pl.multiple_of(c * DC, 256)
                pltpu.sync_copy(
                    acc, enc_hbm.at[pl.ds(row0, RB), pl.ds(col0, DC)]
                )
                return carry2

            lax.fori_loop(0, NCHUNK, body_c, 0)
            return carry

        lax.fori_loop(0, BPW // RB, body_grp, 0)

    return enc_kernel(table_h, x)


def _tc_search(enc, am_pad):
    BB = 128

    def body(enc_ref, am_ref, out_ref):
        am = am_ref[...]
        an = jnp.sqrt(jnp.sum(am * am, axis=1, keepdims=True)) + 1e-12
        am_n = am / an
        e = enc_ref[...]
        en = jnp.sqrt(jnp.sum(e * e, axis=1, keepdims=True)) + 1e-12
        s = lax.dot_general(
            e, am_n, (((1,), (1,)), ((), ())), preferred_element_type=jnp.float32
        )
        out_ref[...] = s / en

    return pl.pallas_call(
        body,
        grid=(B // BB,),
        in_specs=[
            pl.BlockSpec((BB, DP), lambda i: (i, 0)),
            pl.BlockSpec((NUM_CLASSES, DP), lambda i: (0, 0)),
        ],
        out_specs=pl.BlockSpec((BB, NUM_CLASSES), lambda i: (i, 0)),
        out_shape=jax.ShapeDtypeStruct((B, NUM_CLASSES), jnp.float32),
    )(enc, am_pad)


@jax.jit
def kernel(x, id_weight, am_weight):
    table_h = _build_haloed_table(id_weight)
    enc = _sc_encode(table_h, x.astype(jnp.int32))
    am_pad = jnp.pad(am_weight, ((0, 0), (0, DP - D)))
    return _tc_search(enc, am_pad)


# speedup vs baseline: 1.6353x; 1.6353x over previous
import functools

import jax
import jax.numpy as jnp
import numpy as np
from jax import lax
from jax.experimental import pallas as pl
from jax.experimental.pallas import tpu as pltpu
from jax.experimental.pallas import tpu_sc as plsc

B, L, D = 1024, 20, 10000
VOCAB, NUM_CLASSES, NGRAM_N = 1000, 100, 3

NC, NS = 2, 16
NW = NC * NS
BPW = B // NW
RB = 8

NCHUNK = 5
DP = 10240
DC = DP // NCHUNK
HALO = NGRAM_N - 1
W = DC + 128

_NT = L - (NGRAM_N - 1)


def _build_haloed_table(id_weight):
    cols = np.arange(W)[None, :] + (np.arange(NCHUNK) * DC)[:, None] - HALO
    cols = np.where(cols < 0, cols + D, cols)
    wz = jnp.pad(id_weight, ((0, 0), (0, int(cols.max()) + 1 - D)))
    th = jnp.take(wz, jnp.asarray(cols.reshape(-1)), axis=1)
    return th.reshape(VOCAB * NCHUNK, W)


def _sc_encode(table_h, x):
    mesh = plsc.VectorSubcoreMesh(
        core_axis_name="c", subcore_axis_name="s", num_cores=NC, num_subcores=NS
    )

    @functools.partial(
        pl.kernel,
        out_type=jax.ShapeDtypeStruct((B, DP), jnp.float32),
        mesh=mesh,
        compiler_params=pltpu.CompilerParams(use_tc_tiling_on_sc=False),
        scratch_types=[
            pltpu.VMEM((BPW, L), jnp.int32),
            pltpu.VMEM((L,), jnp.int32),
            pltpu.VMEM((L, W), jnp.float32),
            pltpu.VMEM((RB, DC), jnp.float32),
            pltpu.SemaphoreType.DMA,
        ],
    )
    def enc_kernel(table_hbm, x_hbm, enc_hbm, xw, idxv, buf, acc, sem):
        wid = lax.axis_index("s") * NC + lax.axis_index("c")
        base_b = wid * BPW
        pltpu.sync_copy(x_hbm.at[pl.ds(base_b, BPW)], xw)

        def body_grp(i8, carry):
            def body_c(c, carry2):
                def body_r(r, carry3):
                    i = i8 * RB + r
                    idxv[pl.ds(0, 16)] = xw[i, pl.ds(0, 16)] * NCHUNK + c
                    idxv[pl.ds(4, 16)] = xw[i, pl.ds(4, 16)] * NCHUNK + c
                    pltpu.async_copy(table_hbm.at[idxv], buf, sem).wait()

                    for g in range(DC // 16):
                        base = g * 16

                        def tbody(t, a):
                            v = buf[t, pl.ds(base, 16)]
                            v = v * buf[t + 1, pl.ds(base + 1, 16)]
                            v = v * buf[t + 2, pl.ds(base + 2, 16)]
                            return a + v

                        acc[r, pl.ds(base, 16)] = lax.fori_loop(
                            0, _NT, tbody, jnp.zeros((16,), jnp.float32)
                        )
                    return carry3

                lax.fori_loop(0, RB, body_r, 0)
                row0 = pl.multiple_of(base_b + i8 * RB, RB)
                col0 = pl.multiple_of(c * DC, 256)
                pltpu.sync_copy(
                    acc, enc_hbm.at[pl.ds(row0, RB), pl.ds(col0, DC)]
                )
                return carry2

            lax.fori_loop(0, NCHUNK, body_c, 0)
            return carry

        lax.fori_loop(0, BPW // RB, body_grp, 0)

    return enc_kernel(table_h, x)


def _tc_search(enc, am_pad):
    BB = 128

    def body(enc_ref, am_ref, out_ref):
        am = am_ref[...]
        an = jnp.sqrt(jnp.sum(am * am, axis=1, keepdims=True)) + 1e-12
        am_n = am / an
        e = enc_ref[...]
        en = jnp.sqrt(jnp.sum(e * e, axis=1, keepdims=True)) + 1e-12
        s = lax.dot_general(
            e, am_n, (((1,), (1,)), ((), ())), preferred_element_type=jnp.float32
        )
        out_ref[...] = s / en

    return pl.pallas_call(
        body,
        grid=(B // BB,),
        in_specs=[
            pl.BlockSpec((BB, DP), lambda i: (i, 0)),
            pl.BlockSpec((NUM_CLASSES, DP), lambda i: (0, 0)),
        ],
        out_specs=pl.BlockSpec((BB, NUM_CLASSES), lambda i: (i, 0)),
        out_shape=jax.ShapeDtypeStruct((B, NUM_CLASSES), jnp.float32),
    )(enc, am_pad)


@jax.jit
def kernel(x, id_weight, am_weight):
    table_h = _build_haloed_table(id_weight)
    enc = _sc_encode(table_h, x.astype(jnp.int32))
    am_pad = jnp.pad(am_weight, ((0, 0), (0, DP - D)))
    return _tc_search(enc, am_pad)


# speedup vs baseline: 2.2047x; 1.3482x over previous
import functools

import jax
import jax.numpy as jnp
import numpy as np
from jax import lax
from jax.experimental import pallas as pl
from jax.experimental.pallas import tpu as pltpu
from jax.experimental.pallas import tpu_sc as plsc

B, L, D = 1024, 20, 10000
VOCAB, NUM_CLASSES, NGRAM_N = 1000, 100, 3

NC, NS = 2, 16
NW = NC * NS
BPW = B // NW
RB = 8

NCHUNK = 5
DP = 10240
DC = DP // NCHUNK
WB = DC + 32

_NT = L - (NGRAM_N - 1)


def _build_tables(id_weight):
    base = np.arange(WB)[None, :] + (np.arange(NCHUNK) * DC)[:, None]
    exts = []
    for shift in (-2, -1):
        cols = base + shift
        cols = np.where(cols < 0, cols + D, cols)
        exts.append(cols)
    width = int(max(c.max() for c in exts)) + 1
    wz = jnp.pad(id_weight, ((0, 0), (0, width - D))).astype(jnp.bfloat16)
    tabs = []
    for cols in exts:
        th = jnp.take(wz, jnp.asarray(cols.reshape(-1)), axis=1)
        tabs.append(th.reshape(VOCAB * NCHUNK, WB))
    return tabs


def _sc_encode(table_a, table_b, x):
    mesh = plsc.VectorSubcoreMesh(
        core_axis_name="c", subcore_axis_name="s", num_cores=NC, num_subcores=NS
    )

    @functools.partial(
        pl.kernel,
        out_type=jax.ShapeDtypeStruct((B, DP), jnp.bfloat16),
        mesh=mesh,
        compiler_params=pltpu.CompilerParams(use_tc_tiling_on_sc=False),
        scratch_types=[
            pltpu.VMEM((BPW, L), jnp.int32),
            pltpu.VMEM((L,), jnp.int32),
            pltpu.VMEM((L, WB), jnp.bfloat16),
            pltpu.VMEM((L, WB), jnp.bfloat16),
            pltpu.VMEM((RB, DC), jnp.bfloat16),
            pltpu.SemaphoreType.DMA,
        ],
    )
    def enc_kernel(ta_hbm, tb_hbm, x_hbm, enc_hbm, xw, idxv, bufa, bufb, acc, sem):
        wid = lax.axis_index("s") * NC + lax.axis_index("c")
        base_b = wid * BPW
        pltpu.sync_copy(x_hbm.at[pl.ds(base_b, BPW)], xw)

        def body_grp(i8, carry):
            def body_c(c, carry2):
                def body_r(r, carry3):
                    i = i8 * RB + r
                    idxv[pl.ds(0, 16)] = xw[i, pl.ds(0, 16)] * NCHUNK + c
                    idxv[pl.ds(4, 16)] = xw[i, pl.ds(4, 16)] * NCHUNK + c
                    ca = pltpu.async_copy(ta_hbm.at[idxv], bufa, sem)
                    cb = pltpu.async_copy(tb_hbm.at[idxv], bufb, sem)
                    ca.wait()
                    cb.wait()

                    for g in range(DC // 32):
                        base = g * 32

                        def tbody(t, a):
                            v = bufa[t, pl.ds(base, 32)]
                            v = v * bufb[t + 1, pl.ds(base, 32)]
                            v = v * bufa[t + 2, pl.ds(base + 2, 32)]
                            return a + v

                        acc[r, pl.ds(base, 32)] = lax.fori_loop(
                            0, _NT, tbody, jnp.zeros((32,), jnp.bfloat16)
                        )
                    return carry3

                lax.fori_loop(0, RB, body_r, 0)
                row0 = pl.multiple_of(base_b + i8 * RB, RB)
                col0 = pl.multiple_of(c * DC, 256)
                pltpu.sync_copy(
                    acc, enc_hbm.at[pl.ds(row0, RB), pl.ds(col0, DC)]
                )
                return carry2

            lax.fori_loop(0, NCHUNK, body_c, 0)
            return carry

        lax.fori_loop(0, BPW // RB, body_grp, 0)

    return enc_kernel(table_a, table_b, x)


def _tc_search(enc, am_pad):
    BB = 128

    def body(enc_ref, am_ref, out_ref):
        am = am_ref[...]
        an = jnp.sqrt(jnp.sum(am * am, axis=1, keepdims=True)) + 1e-12
        am_n = am / an
        e = enc_ref[...].astype(jnp.float32)
        en = jnp.sqrt(jnp.sum(e * e, axis=1, keepdims=True)) + 1e-12
        s = lax.dot_general(
            e, am_n, (((1,), (1,)), ((), ())), preferred_element_type=jnp.float32
        )
        out_ref[...] = s / en

    return pl.pallas_call(
        body,
        grid=(B // BB,),
        in_specs=[
            pl.BlockSpec((BB, DP), lambda i: (i, 0)),
            pl.BlockSpec((NUM_CLASSES, DP), lambda i: (0, 0)),
        ],
        out_specs=pl.BlockSpec((BB, NUM_CLASSES), lambda i: (i, 0)),
        out_shape=jax.ShapeDtypeStruct((B, NUM_CLASSES), jnp.float32),
    )(enc, am_pad)


@jax.jit
def kernel(x, id_weight, am_weight):
    table_a, table_b = _build_tables(id_weight)
    enc = _sc_encode(table_a, table_b, x.astype(jnp.int32))
    am_pad = jnp.pad(am_weight, ((0, 0), (0, DP - D)))
    return _tc_search(enc, am_pad)


# speedup vs baseline: 2.6377x; 1.1964x over previous
import functools

import jax
import jax.numpy as jnp
import numpy as np
from jax import lax
from jax.experimental import pallas as pl
from jax.experimental.pallas import tpu as pltpu
from jax.experimental.pallas import tpu_sc as plsc

B, L, D = 1024, 20, 10000
VOCAB, NUM_CLASSES, NGRAM_N = 1000, 100, 3

NC, NS = 2, 16
NW = NC * NS
BPW = B // NW
RB = 8

NCHUNK = 5
DP = 10240
DC = DP // NCHUNK
WB = DC + 32

_NT = L - (NGRAM_N - 1)


def _build_tables(id_weight):
    base = np.arange(WB)[None, :] + (np.arange(NCHUNK) * DC)[:, None]
    exts = []
    for shift in (-2, -1):
        cols = base + shift
        cols = np.where(cols < 0, cols + D, cols)
        exts.append(cols)
    width = int(max(c.max() for c in exts)) + 1
    wz = jnp.pad(id_weight, ((0, 0), (0, width - D))).astype(jnp.bfloat16)
    tabs = []
    for cols in exts:
        th = jnp.take(wz, jnp.asarray(cols.reshape(-1)), axis=1)
        tabs.append(th.reshape(VOCAB * NCHUNK, WB))
    return tabs


def _sc_encode(table_a, table_b, x):
    mesh = plsc.VectorSubcoreMesh(
        core_axis_name="c", subcore_axis_name="s", num_cores=NC, num_subcores=NS
    )

    @functools.partial(
        pl.kernel,
        out_type=jax.ShapeDtypeStruct((B, DP), jnp.bfloat16),
        mesh=mesh,
        compiler_params=pltpu.CompilerParams(use_tc_tiling_on_sc=False),
        scratch_types=[
            pltpu.VMEM((BPW, L), jnp.int32),
            pltpu.VMEM((2, L), jnp.int32),
            pltpu.VMEM((2, L, WB), jnp.bfloat16),
            pltpu.VMEM((2, L, WB), jnp.bfloat16),
            pltpu.VMEM((RB, DC), jnp.bfloat16),
            pltpu.SemaphoreType.DMA,
            pltpu.SemaphoreType.DMA,
        ],
    )
    def enc_kernel(
        ta_hbm, tb_hbm, x_hbm, enc_hbm, xw, idxv, bufa, bufb, acc, sem0, sem1
    ):
        wid = lax.axis_index("s") * NC + lax.axis_index("c")
        base_b = wid * BPW
        pltpu.sync_copy(x_hbm.at[pl.ds(base_b, BPW)], xw)
        sems = (sem0, sem1)

        def fire(pb, i, c):
            idxv[pb, pl.ds(0, 16)] = xw[i, pl.ds(0, 16)] * NCHUNK + c
            idxv[pb, pl.ds(4, 16)] = xw[i, pl.ds(4, 16)] * NCHUNK + c
            pltpu.async_copy(ta_hbm.at[idxv.at[pb]], bufa.at[pb], sems[pb])
            pltpu.async_copy(tb_hbm.at[idxv.at[pb]], bufb.at[pb], sems[pb])

        def drain(pb):
            pltpu.make_async_copy(ta_hbm.at[idxv.at[pb]], bufa.at[pb], sems[pb]).wait()
            pltpu.make_async_copy(tb_hbm.at[idxv.at[pb]], bufb.at[pb], sems[pb]).wait()

        def compute(pb, r):
            for g in range(DC // 32):
                base = g * 32

                def tbody(t, a):
                    v = bufa[pb, t, pl.ds(base, 32)]
                    v = v * bufb[pb, t + 1, pl.ds(base, 32)]
                    v = v * bufa[pb, t + 2, pl.ds(base + 2, 32)]
                    return a + v

                acc[r, pl.ds(base, 32)] = lax.fori_loop(
                    0, _NT, tbody, jnp.zeros((32,), jnp.bfloat16)
                )

        def body_grp(i8, carry):
            def body_c(c, carry2):
                fire(0, i8 * RB, c)

                def body_r2(r2, carry3):
                    r0 = r2 * 2
                    fire(1, i8 * RB + r0 + 1, c)
                    drain(0)
                    compute(0, r0)

                    @pl.when(r2 < RB // 2 - 1)
                    def _():
                        fire(0, i8 * RB + r0 + 2, c)

                    drain(1)
                    compute(1, r0 + 1)
                    return carry3

                lax.fori_loop(0, RB // 2, body_r2, 0)
                row0 = pl.multiple_of(base_b + i8 * RB, RB)
                col0 = pl.multiple_of(c * DC, 256)
                pltpu.sync_copy(
                    acc, enc_hbm.at[pl.ds(row0, RB), pl.ds(col0, DC)]
                )
                return carry2

            lax.fori_loop(0, NCHUNK, body_c, 0)
            return carry

        lax.fori_loop(0, BPW // RB, body_grp, 0)

    return enc_kernel(table_a, table_b, x)


def _tc_search(enc, am_pad):
    BB = 128

    def body(enc_ref, am_ref, out_ref):
        am = am_ref[...]
        an = jnp.sqrt(jnp.sum(am * am, axis=1, keepdims=True)) + 1e-12
        am_n = am / an
        e = enc_ref[...].astype(jnp.float32)
        en = jnp.sqrt(jnp.sum(e * e, axis=1, keepdims=True)) + 1e-12
        s = lax.dot_general(
            e, am_n, (((1,), (1,)), ((), ())), preferred_element_type=jnp.float32
        )
        out_ref[...] = s / en

    return pl.pallas_call(
        body,
        grid=(B // BB,),
        in_specs=[
            pl.BlockSpec((BB, DP), lambda i: (i, 0)),
            pl.BlockSpec((NUM_CLASSES, DP), lambda i: (0, 0)),
        ],
        out_specs=pl.BlockSpec((BB, NUM_CLASSES), lambda i: (i, 0)),
        out_shape=jax.ShapeDtypeStruct((B, NUM_CLASSES), jnp.float32),
    )(enc, am_pad)


@jax.jit
def kernel(x, id_weight, am_weight):
    table_a, table_b = _build_tables(id_weight)
    enc = _sc_encode(table_a, table_b, x.astype(jnp.int32))
    am_pad = jnp.pad(am_weight, ((0, 0), (0, DP - D)))
    return _tc_search(enc, am_pad)
